# baseline (speedup 1.0000x reference)
import functools

import jax
import jax.numpy as jnp
from jax import lax
from jax.experimental import pallas as pl
from jax.experimental.pallas import tpu as pltpu
from jax.experimental.pallas import tpu_sc as plsc

VOCAB = 1000000
D = 64
SCALE = 8.0

NC = 2
NS = 16
NW = NC * NS

CHUNK = 128


def _make_emb(B: int):
    assert B % (NW * CHUNK) == 0
    b_per_w = B // NW
    n_chunks = b_per_w // CHUNK
    mesh = plsc.VectorSubcoreMesh(
        core_axis_name="c", subcore_axis_name="s", num_cores=NC, num_subcores=NS
    )

    @functools.partial(
        pl.kernel,
        mesh=mesh,
        out_type=jax.ShapeDtypeStruct((B, D), jnp.float32),
        scratch_types=[
            pltpu.VMEM((CHUNK,), jnp.int32),
            pltpu.VMEM((CHUNK, D), jnp.float32),
            pltpu.SemaphoreType.DMA,
        ],
        compiler_params=pltpu.CompilerParams(use_tc_tiling_on_sc=False),
    )
    def emb(idx_hbm, table_hbm, out_hbm, idx_v, rows_v, sem):
        wid = lax.axis_index("s") * NC + lax.axis_index("c")
        wbase = wid * b_per_w

        def chunk_body(g, carry):
            base = wbase + g * CHUNK
            pltpu.sync_copy(idx_hbm.at[pl.ds(base, CHUNK)], idx_v)
            pltpu.async_copy(table_hbm.at[idx_v], rows_v, sem).wait()

            def scale_row(i, c2):
                for j in range(D // 16):
                    sl = (i, pl.ds(j * 16, 16))
                    rows_v[sl] = rows_v[sl] * SCALE
                return c2

            lax.fori_loop(0, CHUNK, scale_row, 0, unroll=2)
            pltpu.sync_copy(rows_v, out_hbm.at[pl.ds(base, CHUNK)])
            return carry

        lax.fori_loop(0, n_chunks, chunk_body, 0)

    return emb


def kernel(x, table):
    S0, S1 = x.shape
    B = S0 * S1
    idx = x.reshape(B)
    out = _make_emb(B)(idx, table)
    return out.reshape(S0, S1, D)


# speedup vs baseline: 1.2131x; 1.2131x over previous
import functools

import jax
import jax.numpy as jnp
from jax import lax
from jax.experimental import pallas as pl
from jax.experimental.pallas import tpu as pltpu
from jax.experimental.pallas import tpu_sc as plsc

VOCAB = 1000000
D = 64
SCALE = 8.0

NC = 2
NS = 16
NW = NC * NS

CHUNK = 512


def _make_emb(B: int):
    assert B % (NW * 2 * CHUNK) == 0
    b_per_w = B // NW
    n_chunks = b_per_w // CHUNK
    mesh = plsc.VectorSubcoreMesh(
        core_axis_name="c", subcore_axis_name="s", num_cores=NC, num_subcores=NS
    )

    @functools.partial(
        pl.kernel,
        mesh=mesh,
        out_type=jax.ShapeDtypeStruct((B, D), jnp.float32),
        scratch_types=[
            pltpu.VMEM((CHUNK,), jnp.int32),
            pltpu.VMEM((CHUNK,), jnp.int32),
            pltpu.VMEM((CHUNK, D), jnp.float32),
            pltpu.VMEM((CHUNK, D), jnp.float32),
            pltpu.SemaphoreType.DMA,
            pltpu.SemaphoreType.DMA,
            pltpu.SemaphoreType.DMA,
            pltpu.SemaphoreType.DMA,
        ],
        compiler_params=pltpu.CompilerParams(use_tc_tiling_on_sc=False),
    )
    def emb(idx_hbm, table_hbm, out_hbm, idx0, idx1, rows0, rows1,
            gsem0, gsem1, osem0, osem1):
        wid = lax.axis_index("s") * NC + lax.axis_index("c")
        wbase = wid * b_per_w

        idx_v = (idx0, idx1)
        rows_v = (rows0, rows1)
        gsem = (gsem0, gsem1)
        osem = (osem0, osem1)

        def stage_and_gather(g, b):
            base = wbase + g * CHUNK
            pltpu.sync_copy(idx_hbm.at[pl.ds(base, CHUNK)], idx_v[b])
            pltpu.async_copy(table_hbm.at[idx_v[b]], rows_v[b], gsem[b])

        def scale_buf(b):
            def scale_row(i, c2):
                for j in range(D // 16):
                    sl = (i, pl.ds(j * 16, 16))
                    rows_v[b][sl] = rows_v[b][sl] * SCALE
                return c2

            lax.fori_loop(0, CHUNK, scale_row, 0, unroll=4)

        stage_and_gather(0, 0)

        def pair_body(g0, carry):
            for b in (0, 1):
                g = g0 + b
                nb = 1 - b

                @pl.when(g >= 1)
                def _():
                    pltpu.make_async_copy(
                        rows_v[nb],
                        out_hbm.at[pl.ds(wbase + (g - 1) * CHUNK, CHUNK)],
                        osem[nb],
                    ).wait()

                @pl.when(g + 1 < n_chunks)
                def _():
                    stage_and_gather(g + 1, nb)

                pltpu.make_async_copy(
                    table_hbm.at[idx_v[b]], rows_v[b], gsem[b]
                ).wait()
                scale_buf(b)
                pltpu.async_copy(
                    rows_v[b], out_hbm.at[pl.ds(wbase + g * CHUNK, CHUNK)], osem[b]
                )
            return carry

        lax.fori_loop(0, n_chunks // 2, lambda t, c: pair_body(t * 2, c), 0)

        pltpu.make_async_copy(
            rows_v[1],
            out_hbm.at[pl.ds(wbase + (n_chunks - 1) * CHUNK, CHUNK)],
            osem[1],
        ).wait()

    return emb


def kernel(x, table):
    S0, S1 = x.shape
    B = S0 * S1
    idx = x.reshape(B)
    out = _make_emb(B)(idx, table)
    return out.reshape(S0, S1, D)
